# Initial kernel scaffold; baseline (speedup 1.0000x reference)
import jax
import jax.numpy as jnp
from jax.experimental import pallas as pl


def kernel(inputs, pos_emb_table):
    raise NotImplementedError("write your pallas kernel here")


# speedup vs baseline: 1.8054x; 1.8054x over previous
import jax
import jax.numpy as jnp
from jax.experimental import pallas as pl

_BLOCK_S = 512


def _add_body(x_ref, t_ref, o_ref):
    o_ref[...] = x_ref[...] + t_ref[...][None, :, :]


def kernel(inputs, pos_emb_table):
    B, S, D = inputs.shape
    return pl.pallas_call(
        _add_body,
        grid=(S // _BLOCK_S,),
        in_specs=[
            pl.BlockSpec((B, _BLOCK_S, D), lambda i: (0, i, 0)),
            pl.BlockSpec((_BLOCK_S, D), lambda i: (i, 0)),
        ],
        out_specs=pl.BlockSpec((B, _BLOCK_S, D), lambda i: (0, i, 0)),
        out_shape=jax.ShapeDtypeStruct((B, S, D), inputs.dtype),
    )(inputs, pos_emb_table)
